# baseline (speedup 1.0000x reference)
import functools

import jax
import jax.numpy as jnp
from jax import lax
from jax.experimental import pallas as pl
from jax.experimental.pallas import tpu as pltpu
from jax.experimental.pallas import tpu_sc as plsc

N = 100000
D = 128
OUT = 128
S = 10
B = 4096

NC = 2
NS = 16
NW = NC * NS
CH = B // NW
LN = 16

BB = 1024
NBLK = B // BB


def _sc_gather(raw, nidxT, nodes):
    f32, i32 = jnp.float32, jnp.int32
    mesh = plsc.VectorSubcoreMesh(core_axis_name="c", subcore_axis_name="s")
    out_type = [
        jax.ShapeDtypeStruct((B, D), f32),
        jax.ShapeDtypeStruct((B, D), f32),
        jax.ShapeDtypeStruct((B * S, D), f32),
        jax.ShapeDtypeStruct((B * S, D), f32),
    ]
    scratch = [
        pltpu.VMEM((CH,), i32),
        pltpu.VMEM((CH,), i32),
        pltpu.VMEM((S, CH), i32),
        pltpu.VMEM((S, CH), i32),
        pltpu.VMEM((S, CH), i32),
        pltpu.VMEM((CH, D), f32),
        pltpu.VMEM((CH, D), f32),
        pltpu.VMEM((CH, D), f32),
        pltpu.SemaphoreType.DMA,
        pltpu.SemaphoreType.DMA,
        pltpu.SemaphoreType.DMA,
        pltpu.SemaphoreType.DMA,
    ]

    @functools.partial(pl.kernel, mesh=mesh, out_type=out_type,
                       scratch_types=scratch)
    def k(raw_h, nt_h, nodes_h, self_b, total_b, self_n, total_n,
          nbv, cur10, idxs, curall, slots, acc, bufA, bufB,
          sem_self, sem_idx, semA, semB):
        wid = lax.axis_index("s") * NC + lax.axis_index("c")
        base = wid * CH

        pltpu.sync_copy(nodes_h.at[pl.ds(base, CH)], nbv)

        def fire_idx_gathers(node_ref, dst):
            sv = jnp.full((LN,), S, i32)
            for kk in range(CH // LN):
                sl = pl.ds(kk * LN, LN)
                cur10[sl] = node_ref[sl] * sv

            def fire(s, carry):
                offv = jnp.full((LN,), s, i32)
                for kk in range(CH // LN):
                    sl = pl.ds(kk * LN, LN)
                    idxs[s, sl] = cur10[sl] + offv
                pltpu.async_copy(nt_h.at[idxs.at[s]], dst.at[s], sem_idx)
                return carry
            lax.fori_loop(0, S, fire, 0)

            def drain(s, carry):
                pltpu.make_async_copy(nt_h.at[idxs.at[s]], dst.at[s],
                                      sem_idx).wait()
                return carry
            lax.fori_loop(0, S, drain, 0)

        def accumulate(buf):
            def body(r2, carry):
                for dr in range(2):
                    r = r2 * 2 + dr
                    for kk in range(D // LN):
                        sl = pl.ds(kk * LN, LN)
                        plsc.addupdate(acc.at[r, sl], buf[r, sl])
                return carry
            lax.fori_loop(0, CH // 2, body, 0)

        def process(node_ref, self_out, total_out, row0):
            cp_self = pltpu.async_copy(raw_h.at[node_ref], acc, sem_self)
            fire_idx_gathers(node_ref, slots)
            cp_self.wait()
            pltpu.sync_copy(acc, self_out.at[pl.ds(row0, CH)])

            cps = {0: pltpu.async_copy(raw_h.at[slots.at[0]], bufA, semA)}
            for s in range(S):
                buf = bufA if s % 2 == 0 else bufB
                if s + 1 < S:
                    nxt = bufB if s % 2 == 0 else bufA
                    nsem = semB if s % 2 == 0 else semA
                    cps[s + 1] = pltpu.async_copy(
                        raw_h.at[slots.at[s + 1]], nxt, nsem)
                cps[s].wait()
                accumulate(buf)
            pltpu.sync_copy(acc, total_out.at[pl.ds(row0, CH)])

        fire_idx_gathers(nbv, curall)

        process(nbv, self_b, total_b, base)

        def nbody(j, carry):
            process(curall.at[j], self_n, total_n, j * B + base)
            return carry
        lax.fori_loop(0, S, nbody, 0)

    return k(raw, nidxT, nodes)


def _tc_fused(sb, tb, sn, tn, wsa, wsb, w2a, w2b):
    f32 = jnp.float32

    def body(sb_r, tb_r, sn_r, tn_r, wsa_r, wsb_r, w2a_r, w2b_r, out_r,
             h1b_s, acc_s):
        j = pl.program_id(1)
        h1n = jnp.maximum(
            jnp.dot(sn_r[:], wsa_r[:], preferred_element_type=f32)
            + jnp.dot(tn_r[:], wsb_r[:], preferred_element_type=f32), 0.0)

        @pl.when(j == 0)
        def _():
            h1b_s[:] = jnp.maximum(
                jnp.dot(sb_r[:], wsa_r[:], preferred_element_type=f32)
                + jnp.dot(tb_r[:], wsb_r[:], preferred_element_type=f32), 0.0)
            acc_s[:] = h1n

        @pl.when(j > 0)
        def _():
            acc_s[:] = acc_s[:] + h1n

        @pl.when(j == S - 1)
        def _():
            h1b = h1b_s[:]
            out_r[:] = jnp.maximum(
                jnp.dot(h1b, w2a_r[:], preferred_element_type=f32)
                + jnp.dot(acc_s[:] + h1b, w2b_r[:], preferred_element_type=f32),
                0.0)

    return pl.pallas_call(
        body,
        grid=(NBLK, S),
        in_specs=[
            pl.BlockSpec((BB, D), lambda ib, j: (ib, 0)),
            pl.BlockSpec((BB, D), lambda ib, j: (ib, 0)),
            pl.BlockSpec((BB, D), lambda ib, j: (j * NBLK + ib, 0)),
            pl.BlockSpec((BB, D), lambda ib, j: (j * NBLK + ib, 0)),
            pl.BlockSpec((D, OUT), lambda ib, j: (0, 0)),
            pl.BlockSpec((D, OUT), lambda ib, j: (0, 0)),
            pl.BlockSpec((OUT, OUT), lambda ib, j: (0, 0)),
            pl.BlockSpec((OUT, OUT), lambda ib, j: (0, 0)),
        ],
        out_specs=pl.BlockSpec((BB, OUT), lambda ib, j: (ib, 0)),
        out_shape=jax.ShapeDtypeStruct((B, OUT), jnp.float32),
        scratch_shapes=[pltpu.VMEM((BB, OUT), jnp.float32),
                        pltpu.VMEM((BB, OUT), jnp.float32)],
        compiler_params=pltpu.CompilerParams(
            dimension_semantics=("arbitrary", "arbitrary")),
    )(sb, tb, sn, tn, wsa, wsb, w2a, w2b)


def kernel(raw_features, neigh_idx, nodes_batch, W1, W2):
    nflat = neigh_idx.astype(jnp.int32).reshape(-1)
    nodes = nodes_batch.astype(jnp.int32)

    self_b, total_b, self_n, total_n = _sc_gather(raw_features, nflat, nodes)

    inv = 1.0 / (S + 1)
    wsa = W1[:, :D].T
    wsb = W1[:, D:].T * inv
    w2a = W2[:, :OUT].T
    w2b = W2[:, OUT:].T * inv
    return _tc_fused(self_b, total_b, self_n, total_n, wsa, wsb, w2a, w2b)


# speedup vs baseline: 1.0093x; 1.0093x over previous
import functools

import jax
import jax.numpy as jnp
from jax import lax
from jax.experimental import pallas as pl
from jax.experimental.pallas import tpu as pltpu
from jax.experimental.pallas import tpu_sc as plsc

N = 100000
D = 128
OUT = 128
S = 10
B = 4096

NC = 2
NS = 16
NW = NC * NS
CH = B // NW
LN = 16

BB = 1024
NBLK = B // BB


def _sc_gather(raw, nidxT, nodes):
    f32, i32 = jnp.float32, jnp.int32
    mesh = plsc.VectorSubcoreMesh(core_axis_name="c", subcore_axis_name="s")
    out_type = [
        jax.ShapeDtypeStruct((B, D), f32),
        jax.ShapeDtypeStruct((B, D), f32),
        jax.ShapeDtypeStruct((B * S, D), f32),
        jax.ShapeDtypeStruct((B * S, D), f32),
    ]
    scratch = [
        pltpu.VMEM((CH,), i32),
        pltpu.VMEM((CH,), i32),
        pltpu.VMEM((S, CH), i32),
        pltpu.VMEM((S, CH), i32),
        pltpu.VMEM((S, CH), i32),
        pltpu.VMEM((CH, D), f32),
        pltpu.VMEM((CH, D), f32),
        pltpu.VMEM((CH, D), f32),
        pltpu.VMEM((CH, D), f32),
        pltpu.SemaphoreType.DMA,
        pltpu.SemaphoreType.DMA,
        pltpu.SemaphoreType.DMA,
        pltpu.SemaphoreType.DMA,
        pltpu.SemaphoreType.DMA,
    ]

    @functools.partial(pl.kernel, mesh=mesh, out_type=out_type,
                       scratch_types=scratch)
    def k(raw_h, nt_h, nodes_h, self_b, total_b, self_n, total_n,
          nbv, cur10, idxs, curall, slots, acc, sbuf, bufA, bufB,
          sem_self, sem_idx, semA, semB, sem_w):
        wid = lax.axis_index("s") * NC + lax.axis_index("c")
        base = wid * CH

        pltpu.sync_copy(nodes_h.at[pl.ds(base, CH)], nbv)

        def fire_idx_gathers(node_ref, dst):
            sv = jnp.full((LN,), S, i32)
            for kk in range(CH // LN):
                sl = pl.ds(kk * LN, LN)
                cur10[sl] = node_ref[sl] * sv

            def fire(s, carry):
                offv = jnp.full((LN,), s, i32)
                for kk in range(CH // LN):
                    sl = pl.ds(kk * LN, LN)
                    idxs[s, sl] = cur10[sl] + offv
                pltpu.async_copy(nt_h.at[idxs.at[s]], dst.at[s], sem_idx)
                return carry
            lax.fori_loop(0, S, fire, 0)

            def drain(s, carry):
                pltpu.make_async_copy(nt_h.at[idxs.at[s]], dst.at[s],
                                      sem_idx).wait()
                return carry
            lax.fori_loop(0, S, drain, 0)

        def accumulate(buf):
            @plsc.parallel_loop(0, CH, 1, unroll=4)
            def _(r):
                for kk in range(D // LN):
                    sl = pl.ds(kk * LN, LN)
                    plsc.addupdate(acc.at[r, sl], buf[r, sl])

        def process(node_ref, self_out, total_out, row0, drain_prev):
            if drain_prev:
                pltpu.make_async_copy(
                    sbuf, self_out.at[pl.ds(row0, CH)], sem_w).wait()
                pltpu.make_async_copy(
                    acc, total_out.at[pl.ds(row0, CH)], sem_w).wait()
            cp_self = pltpu.async_copy(raw_h.at[node_ref], sbuf, sem_self)
            fire_idx_gathers(node_ref, slots)

            cps = {0: pltpu.async_copy(raw_h.at[slots.at[0]], acc, semA)}
            cps[1] = pltpu.async_copy(raw_h.at[slots.at[1]], bufA, semB)
            cp_self.wait()
            pltpu.async_copy(sbuf, self_out.at[pl.ds(row0, CH)], sem_w)
            cps[0].wait()
            for s in range(1, S):
                buf = bufA if s % 2 == 1 else bufB
                if s + 1 < S:
                    nxt = bufB if s % 2 == 1 else bufA
                    nsem = semA if s % 2 == 1 else semB
                    cps[s + 1] = pltpu.async_copy(
                        raw_h.at[slots.at[s + 1]], nxt, nsem)
                cps[s].wait()
                accumulate(buf)
            accumulate(sbuf)
            pltpu.async_copy(acc, total_out.at[pl.ds(row0, CH)], sem_w)

        fire_idx_gathers(nbv, curall)

        process(nbv, self_b, total_b, base, drain_prev=False)

        def nbody(j, carry):
            process(curall.at[j], self_n, total_n, j * B + base,
                    drain_prev=True)
            return carry
        lax.fori_loop(0, S, nbody, 0)

        pltpu.make_async_copy(sbuf, self_n.at[pl.ds(base, CH)], sem_w).wait()
        pltpu.make_async_copy(acc, total_n.at[pl.ds(base, CH)], sem_w).wait()

    return k(raw, nidxT, nodes)


def _tc_fused(sb, tb, sn, tn, wsa, wsb, w2a, w2b):
    f32 = jnp.float32

    def body(sb_r, tb_r, sn_r, tn_r, wsa_r, wsb_r, w2a_r, w2b_r, out_r,
             h1b_s, acc_s):
        j = pl.program_id(1)
        h1n = jnp.maximum(
            jnp.dot(sn_r[:], wsa_r[:], preferred_element_type=f32)
            + jnp.dot(tn_r[:], wsb_r[:], preferred_element_type=f32), 0.0)

        @pl.when(j == 0)
        def _():
            h1b_s[:] = jnp.maximum(
                jnp.dot(sb_r[:], wsa_r[:], preferred_element_type=f32)
                + jnp.dot(tb_r[:], wsb_r[:], preferred_element_type=f32), 0.0)
            acc_s[:] = h1n

        @pl.when(j > 0)
        def _():
            acc_s[:] = acc_s[:] + h1n

        @pl.when(j == S - 1)
        def _():
            h1b = h1b_s[:]
            out_r[:] = jnp.maximum(
                jnp.dot(h1b, w2a_r[:], preferred_element_type=f32)
                + jnp.dot(acc_s[:] + h1b, w2b_r[:], preferred_element_type=f32),
                0.0)

    return pl.pallas_call(
        body,
        grid=(NBLK, S),
        in_specs=[
            pl.BlockSpec((BB, D), lambda ib, j: (ib, 0)),
            pl.BlockSpec((BB, D), lambda ib, j: (ib, 0)),
            pl.BlockSpec((BB, D), lambda ib, j: (j * NBLK + ib, 0)),
            pl.BlockSpec((BB, D), lambda ib, j: (j * NBLK + ib, 0)),
            pl.BlockSpec((D, OUT), lambda ib, j: (0, 0)),
            pl.BlockSpec((D, OUT), lambda ib, j: (0, 0)),
            pl.BlockSpec((OUT, OUT), lambda ib, j: (0, 0)),
            pl.BlockSpec((OUT, OUT), lambda ib, j: (0, 0)),
        ],
        out_specs=pl.BlockSpec((BB, OUT), lambda ib, j: (ib, 0)),
        out_shape=jax.ShapeDtypeStruct((B, OUT), jnp.float32),
        scratch_shapes=[pltpu.VMEM((BB, OUT), jnp.float32),
                        pltpu.VMEM((BB, OUT), jnp.float32)],
        compiler_params=pltpu.CompilerParams(
            dimension_semantics=("arbitrary", "arbitrary")),
    )(sb, tb, sn, tn, wsa, wsb, w2a, w2b)


def kernel(raw_features, neigh_idx, nodes_batch, W1, W2):
    nflat = neigh_idx.astype(jnp.int32).reshape(-1)
    nodes = nodes_batch.astype(jnp.int32)

    self_b, total_b, self_n, total_n = _sc_gather(raw_features, nflat, nodes)

    inv = 1.0 / (S + 1)
    wsa = W1[:, :D].T
    wsb = W1[:, D:].T * inv
    w2a = W2[:, :OUT].T
    w2b = W2[:, OUT:].T * inv
    return _tc_fused(self_b, total_b, self_n, total_n, wsa, wsb, w2a, w2b)


# speedup vs baseline: 1.2663x; 1.2545x over previous
import functools

import jax
import jax.numpy as jnp
from jax import lax
from jax.experimental import pallas as pl
from jax.experimental.pallas import tpu as pltpu
from jax.experimental.pallas import tpu_sc as plsc

N = 100000
D = 128
OUT = 128
S = 10
B = 4096

NC = 2
NS = 16
NW = NC * NS
CH = B // NW
LN = 16

BB = 1024
NBLK = B // BB


def _sc_gather(raw, nidxT, nodes):
    f32, i32 = jnp.float32, jnp.int32
    mesh = plsc.VectorSubcoreMesh(core_axis_name="c", subcore_axis_name="s")
    out_type = [
        jax.ShapeDtypeStruct((B, D), f32),
        jax.ShapeDtypeStruct((B, D), f32),
        jax.ShapeDtypeStruct((B * S, D), f32),
        jax.ShapeDtypeStruct((B * S, D), f32),
    ]
    scratch = [
        pltpu.VMEM((CH,), i32),
        pltpu.VMEM((S, CH), i32),
        pltpu.VMEM((S, CH), i32),
        pltpu.VMEM((S, CH), i32),
        pltpu.VMEM((CH, D), f32),
        pltpu.VMEM((CH, D), f32),
        pltpu.VMEM((CH, D), f32),
        pltpu.VMEM((CH, D), f32),
        pltpu.SemaphoreType.DMA,
        pltpu.SemaphoreType.DMA,
        pltpu.SemaphoreType.DMA,
        pltpu.SemaphoreType.DMA,
        pltpu.SemaphoreType.DMA,
    ]

    @functools.partial(pl.kernel, mesh=mesh, out_type=out_type,
                       scratch_types=scratch)
    def k(raw_h, nt_h, nodes_h, self_b, total_b, self_n, total_n,
          nbv, idxs, curall, slots, acc, sbuf, bufA, bufB,
          sem_self, sem_idx, semA, semB, sem_w):
        wid = lax.axis_index("s") * NC + lax.axis_index("c")
        base = wid * CH

        pltpu.sync_copy(nodes_h.at[pl.ds(base, CH)], nbv)

        def fire_idx_gathers(node_ref, dst):
            def fire(s, carry):
                offv = jnp.full((LN,), s * N, i32)
                for kk in range(CH // LN):
                    sl = pl.ds(kk * LN, LN)
                    idxs[s, sl] = node_ref[sl] + offv
                pltpu.async_copy(nt_h.at[idxs.at[s]], dst.at[s], sem_idx)
                return carry
            lax.fori_loop(0, S, fire, 0)

            def drain(s, carry):
                pltpu.make_async_copy(nt_h.at[idxs.at[s]], dst.at[s],
                                      sem_idx).wait()
                return carry
            lax.fori_loop(0, S, drain, 0)

        def accumulate(buf):
            @plsc.parallel_loop(0, CH, 1, unroll=4)
            def _(r):
                for kk in range(D // LN):
                    sl = pl.ds(kk * LN, LN)
                    plsc.addupdate(acc.at[r, sl], buf[r, sl])

        def process(node_ref, self_out, total_out, row0, drain_prev):
            if drain_prev:
                pltpu.make_async_copy(
                    sbuf, self_out.at[pl.ds(row0, CH)], sem_w).wait()
                pltpu.make_async_copy(
                    acc, total_out.at[pl.ds(row0, CH)], sem_w).wait()
            cp_self = pltpu.async_copy(raw_h.at[node_ref], sbuf, sem_self)
            fire_idx_gathers(node_ref, slots)

            cps = {0: pltpu.async_copy(raw_h.at[slots.at[0]], acc, semA)}
            cps[1] = pltpu.async_copy(raw_h.at[slots.at[1]], bufA, semB)
            cp_self.wait()
            pltpu.async_copy(sbuf, self_out.at[pl.ds(row0, CH)], sem_w)
            cps[0].wait()
            for s in range(1, S):
                buf = bufA if s % 2 == 1 else bufB
                if s + 1 < S:
                    nxt = bufB if s % 2 == 1 else bufA
                    nsem = semA if s % 2 == 1 else semB
                    cps[s + 1] = pltpu.async_copy(
                        raw_h.at[slots.at[s + 1]], nxt, nsem)
                cps[s].wait()
                accumulate(buf)
            accumulate(sbuf)
            pltpu.async_copy(acc, total_out.at[pl.ds(row0, CH)], sem_w)

        fire_idx_gathers(nbv, curall)

        process(nbv, self_b, total_b, base, drain_prev=False)

        def nbody(j, carry):
            process(curall.at[j], self_n, total_n, j * B + base,
                    drain_prev=True)
            return carry
        lax.fori_loop(0, S, nbody, 0)

        pltpu.make_async_copy(sbuf, self_n.at[pl.ds(base, CH)], sem_w).wait()
        pltpu.make_async_copy(acc, total_n.at[pl.ds(base, CH)], sem_w).wait()

    return k(raw, nidxT, nodes)


def _tc_fused(sb, tb, sn, tn, wsa, wsb, w2a, w2b):
    f32 = jnp.float32

    def body(sb_r, tb_r, sn_r, tn_r, wsa_r, wsb_r, w2a_r, w2b_r, out_r,
             h1b_s, acc_s):
        j = pl.program_id(1)
        h1n = jnp.maximum(
            jnp.dot(sn_r[:], wsa_r[:], preferred_element_type=f32)
            + jnp.dot(tn_r[:], wsb_r[:], preferred_element_type=f32), 0.0)

        @pl.when(j == 0)
        def _():
            h1b_s[:] = jnp.maximum(
                jnp.dot(sb_r[:], wsa_r[:], preferred_element_type=f32)
                + jnp.dot(tb_r[:], wsb_r[:], preferred_element_type=f32), 0.0)
            acc_s[:] = h1n

        @pl.when(j > 0)
        def _():
            acc_s[:] = acc_s[:] + h1n

        @pl.when(j == S - 1)
        def _():
            h1b = h1b_s[:]
            out_r[:] = jnp.maximum(
                jnp.dot(h1b, w2a_r[:], preferred_element_type=f32)
                + jnp.dot(acc_s[:] + h1b, w2b_r[:], preferred_element_type=f32),
                0.0)

    return pl.pallas_call(
        body,
        grid=(NBLK, S),
        in_specs=[
            pl.BlockSpec((BB, D), lambda ib, j: (ib, 0)),
            pl.BlockSpec((BB, D), lambda ib, j: (ib, 0)),
            pl.BlockSpec((BB, D), lambda ib, j: (j * NBLK + ib, 0)),
            pl.BlockSpec((BB, D), lambda ib, j: (j * NBLK + ib, 0)),
            pl.BlockSpec((D, OUT), lambda ib, j: (0, 0)),
            pl.BlockSpec((D, OUT), lambda ib, j: (0, 0)),
            pl.BlockSpec((OUT, OUT), lambda ib, j: (0, 0)),
            pl.BlockSpec((OUT, OUT), lambda ib, j: (0, 0)),
        ],
        out_specs=pl.BlockSpec((BB, OUT), lambda ib, j: (ib, 0)),
        out_shape=jax.ShapeDtypeStruct((B, OUT), jnp.float32),
        scratch_shapes=[pltpu.VMEM((BB, OUT), jnp.float32),
                        pltpu.VMEM((BB, OUT), jnp.float32)],
        compiler_params=pltpu.CompilerParams(
            dimension_semantics=("arbitrary", "arbitrary")),
    )(sb, tb, sn, tn, wsa, wsb, w2a, w2b)


def kernel(raw_features, neigh_idx, nodes_batch, W1, W2):
    nflat = neigh_idx.astype(jnp.int32).T.reshape(-1)
    nodes = nodes_batch.astype(jnp.int32)

    self_b, total_b, self_n, total_n = _sc_gather(raw_features, nflat, nodes)

    inv = 1.0 / (S + 1)
    wsa = W1[:, :D].T
    wsb = W1[:, D:].T * inv
    w2a = W2[:, :OUT].T
    w2b = W2[:, OUT:].T * inv
    return _tc_fused(self_b, total_b, self_n, total_n, wsa, wsb, w2a, w2b)


# speedup vs baseline: 1.3492x; 1.0655x over previous
import functools

import jax
import jax.numpy as jnp
from jax import lax
from jax.experimental import pallas as pl
from jax.experimental.pallas import tpu as pltpu
from jax.experimental.pallas import tpu_sc as plsc

N = 100000
D = 128
OUT = 128
S = 10
B = 4096

NC = 2
NS = 16
NW = NC * NS
CH = B // NW
LN = 16

BB = 1024
NBLK = B // BB


def _sc_gather(raw, nidxT, nodes):
    f32, i32 = jnp.float32, jnp.int32
    mesh = plsc.VectorSubcoreMesh(core_axis_name="c", subcore_axis_name="s")
    out_type = [
        jax.ShapeDtypeStruct((B, D), f32),
        jax.ShapeDtypeStruct((B, D), f32),
        jax.ShapeDtypeStruct((B * S, D), f32),
        jax.ShapeDtypeStruct((B * S, D), f32),
    ]
    scratch = [
        pltpu.VMEM((CH,), i32),
        pltpu.VMEM((S, CH), i32),
        pltpu.VMEM((S, CH), i32),
        pltpu.VMEM((S, CH), i32),
        pltpu.VMEM((CH, D), f32),
        pltpu.VMEM((CH, D), f32),
        pltpu.VMEM((CH, D), f32),
        pltpu.VMEM((CH, D), f32),
        pltpu.SemaphoreType.DMA,
        pltpu.SemaphoreType.DMA,
        pltpu.SemaphoreType.DMA,
        pltpu.SemaphoreType.DMA,
        pltpu.SemaphoreType.DMA,
    ]

    @functools.partial(pl.kernel, mesh=mesh, out_type=out_type,
                       scratch_types=scratch)
    def k(raw_h, nt_h, nodes_h, self_b, total_b, self_n, total_n,
          nbv, idxs, curall, slots, acc, sbuf, bufA, bufB,
          sem_self, sem_idx, semA, semB, sem_w):
        wid = lax.axis_index("s") * NC + lax.axis_index("c")
        base = wid * CH

        pltpu.sync_copy(nodes_h.at[pl.ds(base, CH)], nbv)

        def fire_idx_gathers(node_ref, dst):
            def fire(s, carry):
                offv = jnp.full((LN,), s * N, i32)
                for kk in range(CH // LN):
                    sl = pl.ds(kk * LN, LN)
                    idxs[s, sl] = node_ref[sl] + offv
                pltpu.async_copy(nt_h.at[idxs.at[s]], dst.at[s], sem_idx)
                return carry
            lax.fori_loop(0, S, fire, 0)

            def drain(s, carry):
                pltpu.make_async_copy(nt_h.at[idxs.at[s]], dst.at[s],
                                      sem_idx).wait()
                return carry
            lax.fori_loop(0, S, drain, 0)

        def accumulate(buf):
            @plsc.parallel_loop(0, CH, 1, unroll=4)
            def _(r):
                for kk in range(D // LN):
                    sl = pl.ds(kk * LN, LN)
                    plsc.addupdate(acc.at[r, sl], buf[r, sl])

        def process(node_ref, self_out, total_out, row0, drain_prev):
            if drain_prev:
                pltpu.make_async_copy(
                    sbuf, self_out.at[pl.ds(row0, CH)], sem_w).wait()
                pltpu.make_async_copy(
                    acc, total_out.at[pl.ds(row0, CH)], sem_w).wait()
            cp_self = pltpu.async_copy(raw_h.at[node_ref], sbuf, sem_self)
            fire_idx_gathers(node_ref, slots)

            cps = {0: pltpu.async_copy(raw_h.at[slots.at[0]], acc, semA)}
            cps[1] = pltpu.async_copy(raw_h.at[slots.at[1]], bufA, semB)
            cp_self.wait()
            pltpu.async_copy(sbuf, self_out.at[pl.ds(row0, CH)], sem_w)
            cps[0].wait()
            for s in range(1, S):
                buf = bufA if s % 2 == 1 else bufB
                if s + 1 < S:
                    nxt = bufB if s % 2 == 1 else bufA
                    nsem = semA if s % 2 == 1 else semB
                    cps[s + 1] = pltpu.async_copy(
                        raw_h.at[slots.at[s + 1]], nxt, nsem)
                cps[s].wait()
                accumulate(buf)
            accumulate(sbuf)
            pltpu.async_copy(acc, total_out.at[pl.ds(row0, CH)], sem_w)

        fire_idx_gathers(nbv, curall)

        process(nbv, self_b, total_b, base, drain_prev=False)

        def nbody(j, carry):
            process(curall.at[j], self_n, total_n, j * B + base,
                    drain_prev=True)
            return carry
        lax.fori_loop(0, S, nbody, 0)

        pltpu.make_async_copy(sbuf, self_n.at[pl.ds(base, CH)], sem_w).wait()
        pltpu.make_async_copy(acc, total_n.at[pl.ds(base, CH)], sem_w).wait()

    return k(raw, nidxT, nodes)


def _tc_fused(sb, tb, sn, tn, wsa, wsb, w2a, w2b):
    f32 = jnp.float32

    def body(sb_r, tb_r, sn_r, tn_r, wsa_r, wsb_r, w2a_r, w2b_r, out_r,
             h1b_s, acc_s):
        j = pl.program_id(0)
        h1n = jnp.maximum(
            jnp.dot(sn_r[:], wsa_r[:], preferred_element_type=f32)
            + jnp.dot(tn_r[:], wsb_r[:], preferred_element_type=f32), 0.0)

        @pl.when(j == 0)
        def _():
            h1b_s[:] = jnp.maximum(
                jnp.dot(sb_r[:], wsa_r[:], preferred_element_type=f32)
                + jnp.dot(tb_r[:], wsb_r[:], preferred_element_type=f32), 0.0)
            acc_s[:] = h1n

        @pl.when(j > 0)
        def _():
            acc_s[:] = acc_s[:] + h1n

        @pl.when(j == S - 1)
        def _():
            h1b = h1b_s[:]
            out_r[:] = jnp.maximum(
                jnp.dot(h1b, w2a_r[:], preferred_element_type=f32)
                + jnp.dot(acc_s[:] + h1b, w2b_r[:], preferred_element_type=f32),
                0.0)

    return pl.pallas_call(
        body,
        grid=(S,),
        in_specs=[
            pl.BlockSpec((B, D), lambda j: (0, 0)),
            pl.BlockSpec((B, D), lambda j: (0, 0)),
            pl.BlockSpec((B, D), lambda j: (j, 0)),
            pl.BlockSpec((B, D), lambda j: (j, 0)),
            pl.BlockSpec((D, OUT), lambda j: (0, 0)),
            pl.BlockSpec((D, OUT), lambda j: (0, 0)),
            pl.BlockSpec((OUT, OUT), lambda j: (0, 0)),
            pl.BlockSpec((OUT, OUT), lambda j: (0, 0)),
        ],
        out_specs=pl.BlockSpec((B, OUT), lambda j: (0, 0)),
        out_shape=jax.ShapeDtypeStruct((B, OUT), jnp.float32),
        scratch_shapes=[pltpu.VMEM((B, OUT), jnp.float32),
                        pltpu.VMEM((B, OUT), jnp.float32)],
        compiler_params=pltpu.CompilerParams(
            dimension_semantics=("arbitrary",)),
    )(sb, tb, sn, tn, wsa, wsb, w2a, w2b)


def kernel(raw_features, neigh_idx, nodes_batch, W1, W2):
    nflat = neigh_idx.astype(jnp.int32).T.reshape(-1)
    nodes = nodes_batch.astype(jnp.int32)

    self_b, total_b, self_n, total_n = _sc_gather(raw_features, nflat, nodes)

    inv = 1.0 / (S + 1)
    wsa = W1[:, :D].T
    wsb = W1[:, D:].T * inv
    w2a = W2[:, :OUT].T
    w2b = W2[:, OUT:].T * inv
    return _tc_fused(self_b, total_b, self_n, total_n, wsa, wsb, w2a, w2b)


# speedup vs baseline: 1.3971x; 1.0354x over previous
import functools

import jax
import jax.numpy as jnp
from jax import lax
from jax.experimental import pallas as pl
from jax.experimental.pallas import tpu as pltpu
from jax.experimental.pallas import tpu_sc as plsc

N = 100000
D = 128
OUT = 128
S = 10
B = 4096

NC = 2
NS = 16
NW = NC * NS
CH = B // NW
LN = 16

BB = 1024
NBLK = B // BB


def _sc_gather(raw, nidxT, nodes):
    f32, i32 = jnp.float32, jnp.int32
    mesh = plsc.VectorSubcoreMesh(core_axis_name="c", subcore_axis_name="s")
    out_type = [
        jax.ShapeDtypeStruct((B, D), f32),
        jax.ShapeDtypeStruct((B, D), f32),
        jax.ShapeDtypeStruct((B * S, D), f32),
        jax.ShapeDtypeStruct((B * S, D), f32),
    ]
    scratch = [
        pltpu.VMEM((CH,), i32),
        pltpu.VMEM((S, CH), i32),
        pltpu.VMEM((S, CH), i32),
        pltpu.VMEM((S, CH), i32),
        pltpu.VMEM((S, CH), i32),
        pltpu.VMEM((S, CH), i32),
        pltpu.VMEM((CH, D), f32),
        pltpu.VMEM((CH, D), f32),
        pltpu.VMEM((CH, D), f32),
        pltpu.VMEM((CH, D), f32),
        pltpu.SemaphoreType.DMA,
        pltpu.SemaphoreType.DMA,
        pltpu.SemaphoreType.DMA,
        pltpu.SemaphoreType.DMA,
        pltpu.SemaphoreType.DMA,
    ]

    @functools.partial(pl.kernel, mesh=mesh, out_type=out_type,
                       scratch_types=scratch)
    def k(raw_h, nt_h, nodes_h, self_b, total_b, self_n, total_n,
          nbv, idxs0, idxs1, curall, slots0, slots1, acc, sbuf, bufA, bufB,
          sem_self, sem_idx, semA, semB, sem_w):
        wid = lax.axis_index("s") * NC + lax.axis_index("c")
        base = wid * CH

        pltpu.sync_copy(nodes_h.at[pl.ds(base, CH)], nbv)

        def fire_idx(node_ref, idxs, dst):
            def fire(s, carry):
                offv = jnp.full((LN,), s * N, i32)
                for kk in range(CH // LN):
                    sl = pl.ds(kk * LN, LN)
                    idxs[s, sl] = node_ref[sl] + offv
                pltpu.async_copy(nt_h.at[idxs.at[s]], dst.at[s], sem_idx)
                return carry
            lax.fori_loop(0, S, fire, 0)

        def drain_idx(idxs, dst):
            def drain(s, carry):
                pltpu.make_async_copy(nt_h.at[idxs.at[s]], dst.at[s],
                                      sem_idx).wait()
                return carry
            lax.fori_loop(0, S, drain, 0)

        def accumulate(buf):
            @plsc.parallel_loop(0, CH, 1, unroll=4)
            def _(r):
                for kk in range(D // LN):
                    sl = pl.ds(kk * LN, LN)
                    plsc.addupdate(acc.at[r, sl], buf[r, sl])

        def process(node_ref, my_idxs, my_slots, pre_node_ref, pre_idxs,
                    pre_slots, self_out, total_out, row0, drain_prev):
            if drain_prev:
                pltpu.make_async_copy(
                    sbuf, self_out.at[pl.ds(row0, CH)], sem_w).wait()
                pltpu.make_async_copy(
                    acc, total_out.at[pl.ds(row0, CH)], sem_w).wait()
            drain_idx(my_idxs, my_slots)

            cps = {0: pltpu.async_copy(raw_h.at[my_slots.at[0]], acc, semA)}
            cps[1] = pltpu.async_copy(raw_h.at[my_slots.at[1]], bufA, semB)
            cp_self = pltpu.async_copy(raw_h.at[node_ref], sbuf, sem_self)
            if pre_node_ref is not None:
                fire_idx(pre_node_ref, pre_idxs, pre_slots)
            cp_self.wait()
            pltpu.async_copy(sbuf, self_out.at[pl.ds(row0, CH)], sem_w)
            cps[0].wait()
            for s in range(1, S):
                buf = bufA if s % 2 == 1 else bufB
                if s + 1 < S:
                    nxt = bufB if s % 2 == 1 else bufA
                    nsem = semA if s % 2 == 1 else semB
                    cps[s + 1] = pltpu.async_copy(
                        raw_h.at[my_slots.at[s + 1]], nxt, nsem)
                cps[s].wait()
                accumulate(buf)
            accumulate(sbuf)
            pltpu.async_copy(acc, total_out.at[pl.ds(row0, CH)], sem_w)

        fire_idx(nbv, idxs0, curall)
        drain_idx(idxs0, curall)

        fire_idx(nbv, idxs0, slots0)
        process(nbv, idxs0, slots0, curall.at[0], idxs1, slots1,
                self_b, total_b, base, drain_prev=False)

        def pair(jj, carry):
            c0 = 2 * jj
            c1 = 2 * jj + 1
            nxt1 = jnp.minimum(c1 + 1, S - 1)
            process(curall.at[c0], idxs1, slots1, curall.at[c1],
                    idxs0, slots0, self_n, total_n, c0 * B + base,
                    drain_prev=True)
            process(curall.at[c1], idxs0, slots0, curall.at[nxt1],
                    idxs1, slots1, self_n, total_n, c1 * B + base,
                    drain_prev=True)
            return carry
        lax.fori_loop(0, S // 2, pair, 0)

        drain_idx(idxs1, slots1)
        pltpu.make_async_copy(sbuf, self_n.at[pl.ds(base, CH)], sem_w).wait()
        pltpu.make_async_copy(acc, total_n.at[pl.ds(base, CH)], sem_w).wait()

    return k(raw, nidxT, nodes)


def _tc_fused(sb, tb, sn, tn, wsa, wsb, w2a, w2b):
    f32 = jnp.float32

    def body(sb_r, tb_r, sn_r, tn_r, wsa_r, wsb_r, w2a_r, w2b_r, out_r,
             h1b_s, acc_s):
        j = pl.program_id(0)
        h1n = jnp.maximum(
            jnp.dot(sn_r[:], wsa_r[:], preferred_element_type=f32)
            + jnp.dot(tn_r[:], wsb_r[:], preferred_element_type=f32), 0.0)

        @pl.when(j == 0)
        def _():
            h1b_s[:] = jnp.maximum(
                jnp.dot(sb_r[:], wsa_r[:], preferred_element_type=f32)
                + jnp.dot(tb_r[:], wsb_r[:], preferred_element_type=f32), 0.0)
            acc_s[:] = h1n

        @pl.when(j > 0)
        def _():
            acc_s[:] = acc_s[:] + h1n

        @pl.when(j == S - 1)
        def _():
            h1b = h1b_s[:]
            out_r[:] = jnp.maximum(
                jnp.dot(h1b, w2a_r[:], preferred_element_type=f32)
                + jnp.dot(acc_s[:] + h1b, w2b_r[:], preferred_element_type=f32),
                0.0)

    return pl.pallas_call(
        body,
        grid=(S,),
        in_specs=[
            pl.BlockSpec((B, D), lambda j: (0, 0)),
            pl.BlockSpec((B, D), lambda j: (0, 0)),
            pl.BlockSpec((B, D), lambda j: (j, 0)),
            pl.BlockSpec((B, D), lambda j: (j, 0)),
            pl.BlockSpec((D, OUT), lambda j: (0, 0)),
            pl.BlockSpec((D, OUT), lambda j: (0, 0)),
            pl.BlockSpec((OUT, OUT), lambda j: (0, 0)),
            pl.BlockSpec((OUT, OUT), lambda j: (0, 0)),
        ],
        out_specs=pl.BlockSpec((B, OUT), lambda j: (0, 0)),
        out_shape=jax.ShapeDtypeStruct((B, OUT), jnp.float32),
        scratch_shapes=[pltpu.VMEM((B, OUT), jnp.float32),
                        pltpu.VMEM((B, OUT), jnp.float32)],
        compiler_params=pltpu.CompilerParams(
            dimension_semantics=("arbitrary",)),
    )(sb, tb, sn, tn, wsa, wsb, w2a, w2b)


def kernel(raw_features, neigh_idx, nodes_batch, W1, W2):
    nflat = neigh_idx.astype(jnp.int32).T.reshape(-1)
    nodes = nodes_batch.astype(jnp.int32)

    self_b, total_b, self_n, total_n = _sc_gather(raw_features, nflat, nodes)

    inv = 1.0 / (S + 1)
    wsa = W1[:, :D].T
    wsb = W1[:, D:].T * inv
    w2a = W2[:, :OUT].T
    w2b = W2[:, OUT:].T * inv
    return _tc_fused(self_b, total_b, self_n, total_n, wsa, wsb, w2a, w2b)


# speedup vs baseline: 1.4179x; 1.0149x over previous
import functools

import jax
import jax.numpy as jnp
from jax import lax
from jax.experimental import pallas as pl
from jax.experimental.pallas import tpu as pltpu
from jax.experimental.pallas import tpu_sc as plsc

N = 100000
D = 128
OUT = 128
S = 10
B = 4096

NC = 2
NS = 16
NW = NC * NS
CH = B // NW
LN = 16

BB = 1024
NBLK = B // BB


def _sc_gather(raw, nidxT, nodes):
    f32, i32 = jnp.float32, jnp.int32
    mesh = plsc.VectorSubcoreMesh(core_axis_name="c", subcore_axis_name="s")
    out_type = [
        jax.ShapeDtypeStruct((B, D), f32),
        jax.ShapeDtypeStruct((B, D), f32),
        jax.ShapeDtypeStruct((B * S, D), f32),
        jax.ShapeDtypeStruct((B * S, D), f32),
    ]
    scratch = [
        pltpu.VMEM((CH,), i32),
        pltpu.VMEM((S, CH), i32),
        pltpu.VMEM((S, CH), i32),
        pltpu.VMEM((S, CH), i32),
        pltpu.VMEM((S, CH), i32),
        pltpu.VMEM((S, CH), i32),
        pltpu.VMEM((CH, D), f32),
        pltpu.VMEM((CH, D), f32),
        pltpu.VMEM((CH, D), f32),
        pltpu.VMEM((CH, D), f32),
        pltpu.VMEM((CH, D), f32),
        pltpu.SemaphoreType.DMA,
        pltpu.SemaphoreType.DMA,
        pltpu.SemaphoreType.DMA,
        pltpu.SemaphoreType.DMA,
        pltpu.SemaphoreType.DMA,
        pltpu.SemaphoreType.DMA,
        pltpu.SemaphoreType.DMA,
    ]

    @functools.partial(pl.kernel, mesh=mesh, out_type=out_type,
                       scratch_types=scratch)
    def k(raw_h, nt_h, nodes_h, self_b, total_b, self_n, total_n,
          nbv, idxs0, idxs1, curall, slots0, slots1, acc, sbuf,
          bufA, bufB, bufC,
          sem_self, sem_idx, sem0, semA, semB, semC, sem_w):
        wid = lax.axis_index("s") * NC + lax.axis_index("c")
        base = wid * CH

        pltpu.sync_copy(nodes_h.at[pl.ds(base, CH)], nbv)

        def fire_idx(node_ref, idxs, dst):
            def fire(s, carry):
                offv = jnp.full((LN,), s * N, i32)
                for kk in range(CH // LN):
                    sl = pl.ds(kk * LN, LN)
                    idxs[s, sl] = node_ref[sl] + offv
                pltpu.async_copy(nt_h.at[idxs.at[s]], dst.at[s], sem_idx)
                return carry
            lax.fori_loop(0, S, fire, 0)

        def drain_idx(idxs, dst):
            def drain(s, carry):
                pltpu.make_async_copy(nt_h.at[idxs.at[s]], dst.at[s],
                                      sem_idx).wait()
                return carry
            lax.fori_loop(0, S, drain, 0)

        def accumulate(buf):
            @plsc.parallel_loop(0, CH, 1, unroll=4)
            def _(r):
                for kk in range(D // LN):
                    sl = pl.ds(kk * LN, LN)
                    plsc.addupdate(acc.at[r, sl], buf[r, sl])

        def process(node_ref, my_idxs, my_slots, pre_node_ref, pre_idxs,
                    pre_slots, self_out, total_out, row0, drain_prev):
            if drain_prev:
                pltpu.make_async_copy(
                    sbuf, self_out.at[pl.ds(row0, CH)], sem_w).wait()
                pltpu.make_async_copy(
                    acc, total_out.at[pl.ds(row0, CH)], sem_w).wait()
            drain_idx(my_idxs, my_slots)

            bufs = [bufA, bufB, bufC]
            sems = [semA, semB, semC]
            cps = {0: pltpu.async_copy(raw_h.at[my_slots.at[0]], acc, sem0)}
            cps[1] = pltpu.async_copy(raw_h.at[my_slots.at[1]], bufA, semA)
            cps[2] = pltpu.async_copy(raw_h.at[my_slots.at[2]], bufB, semB)
            cp_self = pltpu.async_copy(raw_h.at[node_ref], sbuf, sem_self)
            if pre_node_ref is not None:
                fire_idx(pre_node_ref, pre_idxs, pre_slots)
            cp_self.wait()
            pltpu.async_copy(sbuf, self_out.at[pl.ds(row0, CH)], sem_w)
            cps[0].wait()
            for s in range(1, S):
                p = (s - 1) % 3
                if s + 2 < S:
                    q = (s + 1) % 3
                    cps[s + 2] = pltpu.async_copy(
                        raw_h.at[my_slots.at[s + 2]], bufs[q], sems[q])
                cps[s].wait()
                accumulate(bufs[p])
            accumulate(sbuf)
            pltpu.async_copy(acc, total_out.at[pl.ds(row0, CH)], sem_w)

        fire_idx(nbv, idxs0, curall)
        drain_idx(idxs0, curall)

        fire_idx(nbv, idxs0, slots0)
        process(nbv, idxs0, slots0, curall.at[0], idxs1, slots1,
                self_b, total_b, base, drain_prev=False)

        def pair(jj, carry):
            c0 = 2 * jj
            c1 = 2 * jj + 1
            nxt1 = jnp.minimum(c1 + 1, S - 1)
            process(curall.at[c0], idxs1, slots1, curall.at[c1],
                    idxs0, slots0, self_n, total_n, c0 * B + base,
                    drain_prev=True)
            process(curall.at[c1], idxs0, slots0, curall.at[nxt1],
                    idxs1, slots1, self_n, total_n, c1 * B + base,
                    drain_prev=True)
            return carry
        lax.fori_loop(0, S // 2, pair, 0)

        drain_idx(idxs1, slots1)
        pltpu.make_async_copy(sbuf, self_n.at[pl.ds(base, CH)], sem_w).wait()
        pltpu.make_async_copy(acc, total_n.at[pl.ds(base, CH)], sem_w).wait()

    return k(raw, nidxT, nodes)


def _tc_fused(sb, tb, sn, tn, wsa, wsb, w2a, w2b):
    f32 = jnp.float32

    def body(sb_r, tb_r, sn_r, tn_r, wsa_r, wsb_r, w2a_r, w2b_r, out_r,
             h1b_s, acc_s):
        j = pl.program_id(0)
        h1n = jnp.maximum(
            jnp.dot(sn_r[:], wsa_r[:], preferred_element_type=f32)
            + jnp.dot(tn_r[:], wsb_r[:], preferred_element_type=f32), 0.0)

        @pl.when(j == 0)
        def _():
            h1b_s[:] = jnp.maximum(
                jnp.dot(sb_r[:], wsa_r[:], preferred_element_type=f32)
                + jnp.dot(tb_r[:], wsb_r[:], preferred_element_type=f32), 0.0)
            acc_s[:] = h1n

        @pl.when(j > 0)
        def _():
            acc_s[:] = acc_s[:] + h1n

        @pl.when(j == S - 1)
        def _():
            h1b = h1b_s[:]
            out_r[:] = jnp.maximum(
                jnp.dot(h1b, w2a_r[:], preferred_element_type=f32)
                + jnp.dot(acc_s[:] + h1b, w2b_r[:], preferred_element_type=f32),
                0.0)

    return pl.pallas_call(
        body,
        grid=(S,),
        in_specs=[
            pl.BlockSpec((B, D), lambda j: (0, 0)),
            pl.BlockSpec((B, D), lambda j: (0, 0)),
            pl.BlockSpec((B, D), lambda j: (j, 0)),
            pl.BlockSpec((B, D), lambda j: (j, 0)),
            pl.BlockSpec((D, OUT), lambda j: (0, 0)),
            pl.BlockSpec((D, OUT), lambda j: (0, 0)),
            pl.BlockSpec((OUT, OUT), lambda j: (0, 0)),
            pl.BlockSpec((OUT, OUT), lambda j: (0, 0)),
        ],
        out_specs=pl.BlockSpec((B, OUT), lambda j: (0, 0)),
        out_shape=jax.ShapeDtypeStruct((B, OUT), jnp.float32),
        scratch_shapes=[pltpu.VMEM((B, OUT), jnp.float32),
                        pltpu.VMEM((B, OUT), jnp.float32)],
        compiler_params=pltpu.CompilerParams(
            dimension_semantics=("arbitrary",)),
    )(sb, tb, sn, tn, wsa, wsb, w2a, w2b)


def kernel(raw_features, neigh_idx, nodes_batch, W1, W2):
    nflat = neigh_idx.astype(jnp.int32).T.reshape(-1)
    nodes = nodes_batch.astype(jnp.int32)

    self_b, total_b, self_n, total_n = _sc_gather(raw_features, nflat, nodes)

    inv = 1.0 / (S + 1)
    wsa = W1[:, :D].T
    wsb = W1[:, D:].T * inv
    w2a = W2[:, :OUT].T
    w2b = W2[:, OUT:].T * inv
    return _tc_fused(self_b, total_b, self_n, total_n, wsa, wsb, w2a, w2b)
